# baseline (speedup 1.0000x reference)
import functools

import jax
import jax.numpy as jnp
from jax import lax
from jax.experimental import pallas as pl
from jax.experimental.pallas import tpu as pltpu
from jax.experimental.pallas import tpu_sc as plsc

N_LABELS = 1000
N_DEPTHS = 4
BATCH = 16384
D_WORDS = 250
D_PAD = 256

NUM_CORES = 2
NUM_SUBCORES = 16
NUM_WORKERS = NUM_CORES * NUM_SUBCORES
B_PER_W = BATCH // NUM_WORKERS
CHUNK = 128
N_CHUNKS = B_PER_W // CHUNK
TOTAL_CHUNKS = N_DEPTHS * N_CHUNKS

BLK = 256


def _make_sc_gather():
  mesh = plsc.VectorSubcoreMesh(core_axis_name="c", subcore_axis_name="s")

  @functools.partial(
      pl.kernel,
      mesh=mesh,
      out_type=[jax.ShapeDtypeStruct((BATCH, D_PAD), jnp.int32)
                for _ in range(N_DEPTHS)],
      scratch_types=[
          pltpu.VMEM((B_PER_W,), jnp.int32),
          pltpu.VMEM((2, CHUNK), jnp.int32),
          pltpu.VMEM((2, CHUNK, D_PAD), jnp.int32),
          pltpu.SemaphoreType.DMA,
          pltpu.SemaphoreType.DMA,
      ],
  )
  def gather_kernel(table, labels, out0, out1, out2, out3,
                    lab_v, idx_v, rows_v, sem0, sem1):
    outs = (out0, out1, out2, out3)
    sems = (sem0, sem1)
    wid = lax.axis_index("s") * NUM_CORES + lax.axis_index("c")
    base = wid * B_PER_W

    pltpu.sync_copy(labels.at[pl.ds(base, B_PER_W)], lab_v)

    def fill_idx(g):
      d, c = divmod(g, N_CHUNKS)
      buf = g % 2
      off = jnp.int32(d * N_LABELS)
      for i in range(CHUNK // 16):
        sl = pl.ds(c * CHUNK + i * 16, 16)
        idx_v[buf, pl.ds(i * 16, 16)] = lab_v[sl] + off

    def start_gather(g):
      buf = g % 2
      copy = pltpu.make_async_copy(
          table.at[idx_v.at[buf]], rows_v.at[buf], sems[buf])
      copy.start()
      return copy

    def drain(g, copy):
      d, c = divmod(g, N_CHUNKS)
      buf = g % 2
      copy.wait()
      pltpu.sync_copy(rows_v.at[buf],
                      outs[d].at[pl.ds(base + c * CHUNK, CHUNK)])

    fill_idx(0)
    inflight = start_gather(0)
    for g in range(1, TOTAL_CHUNKS):
      fill_idx(g)
      nxt = start_gather(g)
      drain(g - 1, inflight)
      inflight = nxt
    drain(TOTAL_CHUNKS - 1, inflight)

  return gather_kernel


_sc_gather = _make_sc_gather()


def _unpack_tc_kernel(o_ref, out_ref):
  w = jnp.transpose(o_ref[...].astype(jnp.uint32))
  planes = [(w >> (8 * m)) & 1 for m in range(4)]
  x = jnp.stack(planes, axis=1)
  x = x.reshape(4 * D_PAD, BLK)
  out_ref[...] = x[:N_LABELS, :].astype(jnp.int8)


_unpack_tc = pl.pallas_call(
    _unpack_tc_kernel,
    grid=(BATCH // BLK,),
    in_specs=[pl.BlockSpec((BLK, D_PAD), lambda j: (j, 0))],
    out_specs=pl.BlockSpec((N_LABELS, BLK), lambda j: (0, j)),
    out_shape=jax.ShapeDtypeStruct((N_LABELS, BATCH), jnp.int8),
)


@jax.jit
def kernel(labels, adversaries):
  a = jnp.pad(adversaries, ((0, 0), (0, 0), (0, 4 * D_PAD - N_LABELS)))
  a = a.reshape(N_DEPTHS, N_LABELS, D_PAD, 4).astype(jnp.uint32)
  w = (a[..., 0] | (a[..., 1] << 8) | (a[..., 2] << 16) | (a[..., 3] << 24))
  tbl = jax.lax.bitcast_convert_type(
      w.reshape(N_DEPTHS * N_LABELS, D_PAD), jnp.int32)
  lab = labels.reshape(BATCH)

  outs = _sc_gather(tbl, lab)
  return tuple(jnp.transpose(_unpack_tc(o)) != 0 for o in outs)


# speedup vs baseline: 1.2557x; 1.2557x over previous
import functools

import jax
import jax.numpy as jnp
from jax import lax
from jax.experimental import pallas as pl
from jax.experimental.pallas import tpu as pltpu
from jax.experimental.pallas import tpu_sc as plsc

N_LABELS = 1000
N_DEPTHS = 4
BATCH = 16384
D_WORDS = 250
D_PAD = 256

NUM_CORES = 2
NUM_SUBCORES = 16
NUM_WORKERS = NUM_CORES * NUM_SUBCORES
B_PER_W = BATCH // NUM_WORKERS
CHUNK = 128
N_CHUNKS = B_PER_W // CHUNK
TOTAL_CHUNKS = N_DEPTHS * N_CHUNKS

BLK = 256


def _make_sc_gather():
  mesh = plsc.VectorSubcoreMesh(core_axis_name="c", subcore_axis_name="s")

  @functools.partial(
      pl.kernel,
      mesh=mesh,
      out_type=[jax.ShapeDtypeStruct((BATCH, D_PAD), jnp.int32)
                for _ in range(N_DEPTHS)],
      scratch_types=[
          pltpu.VMEM((B_PER_W,), jnp.int32),
          pltpu.VMEM((2, CHUNK), jnp.int32),
          pltpu.VMEM((2, CHUNK, D_PAD), jnp.int32),
          pltpu.SemaphoreType.DMA,
          pltpu.SemaphoreType.DMA,
      ],
  )
  def gather_kernel(table, labels, out0, out1, out2, out3,
                    lab_v, idx_v, rows_v, sem0, sem1):
    outs = (out0, out1, out2, out3)
    sems = (sem0, sem1)
    wid = lax.axis_index("s") * NUM_CORES + lax.axis_index("c")
    base = wid * B_PER_W

    pltpu.sync_copy(labels.at[pl.ds(base, B_PER_W)], lab_v)

    def fill_idx(g):
      d, c = divmod(g, N_CHUNKS)
      buf = g % 2
      off = jnp.int32(d * N_LABELS)
      for i in range(CHUNK // 16):
        sl = pl.ds(c * CHUNK + i * 16, 16)
        idx_v[buf, pl.ds(i * 16, 16)] = lab_v[sl] + off

    def start_gather(g):
      buf = g % 2
      copy = pltpu.make_async_copy(
          table.at[idx_v.at[buf]], rows_v.at[buf], sems[buf])
      copy.start()
      return copy

    def drain(g, copy):
      d, c = divmod(g, N_CHUNKS)
      buf = g % 2
      copy.wait()
      pltpu.sync_copy(rows_v.at[buf],
                      outs[d].at[pl.ds(base + c * CHUNK, CHUNK)])

    fill_idx(0)
    inflight = start_gather(0)
    for g in range(1, TOTAL_CHUNKS):
      fill_idx(g)
      nxt = start_gather(g)
      drain(g - 1, inflight)
      inflight = nxt
    drain(TOTAL_CHUNKS - 1, inflight)

  return gather_kernel


_sc_gather = _make_sc_gather()


def _unpack_tc_kernel(o_ref, out_ref):
  w = jnp.transpose(o_ref[...])
  x = pltpu.bitcast(w, jnp.int8)
  out_ref[...] = x[:N_LABELS, :]


_unpack_tc = pl.pallas_call(
    _unpack_tc_kernel,
    grid=(BATCH // BLK,),
    in_specs=[pl.BlockSpec((BLK, D_PAD), lambda j: (j, 0))],
    out_specs=pl.BlockSpec((N_LABELS, BLK), lambda j: (0, j)),
    out_shape=jax.ShapeDtypeStruct((N_LABELS, BATCH), jnp.int8),
)


@jax.jit
def kernel(labels, adversaries):
  a = jnp.pad(adversaries, ((0, 0), (0, 0), (0, 4 * D_PAD - N_LABELS)))
  a = a.reshape(N_DEPTHS, N_LABELS, D_PAD, 4).astype(jnp.uint32)
  w = (a[..., 0] | (a[..., 1] << 8) | (a[..., 2] << 16) | (a[..., 3] << 24))
  tbl = jax.lax.bitcast_convert_type(
      w.reshape(N_DEPTHS * N_LABELS, D_PAD), jnp.int32)
  lab = labels.reshape(BATCH)

  outs = _sc_gather(tbl, lab)
  return tuple(jnp.transpose(_unpack_tc(o)) != 0 for o in outs)


# speedup vs baseline: 1.9601x; 1.5609x over previous
import functools

import jax
import jax.numpy as jnp
from jax import lax
from jax.experimental import pallas as pl
from jax.experimental.pallas import tpu as pltpu
from jax.experimental.pallas import tpu_sc as plsc

N_LABELS = 1000
N_DEPTHS = 4
BATCH = 16384
D_WORDS = 250
D_PAD = 256

NUM_CORES = 2
NUM_SUBCORES = 16
NUM_WORKERS = NUM_CORES * NUM_SUBCORES
B_PER_W = BATCH // NUM_WORKERS
CHUNK = 128
N_CHUNKS = B_PER_W // CHUNK
TOTAL_CHUNKS = N_DEPTHS * N_CHUNKS

BLK = 512


def _make_sc_gather():
  mesh = plsc.VectorSubcoreMesh(core_axis_name="c", subcore_axis_name="s")

  @functools.partial(
      pl.kernel,
      mesh=mesh,
      out_type=[jax.ShapeDtypeStruct((BATCH, D_PAD), jnp.int32)
                for _ in range(N_DEPTHS)],
      scratch_types=[
          pltpu.VMEM((B_PER_W,), jnp.int32),
          pltpu.VMEM((2, CHUNK), jnp.int32),
          pltpu.VMEM((2, CHUNK, D_PAD), jnp.int32),
          pltpu.SemaphoreType.DMA,
          pltpu.SemaphoreType.DMA,
      ],
  )
  def gather_kernel(table, labels, out0, out1, out2, out3,
                    lab_v, idx_v, rows_v, sem0, sem1):
    outs = (out0, out1, out2, out3)
    sems = (sem0, sem1)
    wid = lax.axis_index("s") * NUM_CORES + lax.axis_index("c")
    base = wid * B_PER_W

    pltpu.sync_copy(labels.at[pl.ds(base, B_PER_W)], lab_v)

    def fill_idx(g):
      d, c = divmod(g, N_CHUNKS)
      buf = g % 2
      off = jnp.int32(d * N_LABELS)
      for i in range(CHUNK // 16):
        sl = pl.ds(c * CHUNK + i * 16, 16)
        idx_v[buf, pl.ds(i * 16, 16)] = lab_v[sl] + off

    def start_gather(g):
      buf = g % 2
      copy = pltpu.make_async_copy(
          table.at[idx_v.at[buf]], rows_v.at[buf], sems[buf])
      copy.start()
      return copy

    def drain(g, copy):
      d, c = divmod(g, N_CHUNKS)
      buf = g % 2
      copy.wait()
      pltpu.sync_copy(rows_v.at[buf],
                      outs[d].at[pl.ds(base + c * CHUNK, CHUNK)])

    fill_idx(0)
    inflight = start_gather(0)
    for g in range(1, TOTAL_CHUNKS):
      fill_idx(g)
      nxt = start_gather(g)
      drain(g - 1, inflight)
      inflight = nxt
    drain(TOTAL_CHUNKS - 1, inflight)

  return gather_kernel


_sc_gather = _make_sc_gather()


def _unpack_tc_kernel(*refs):
  o_refs, out_refs = refs[:N_DEPTHS], refs[N_DEPTHS:]
  for o_ref, out_ref in zip(o_refs, out_refs):
    w = jnp.transpose(o_ref[...])
    x = pltpu.bitcast(w, jnp.int8)
    out_ref[...] = x[:N_LABELS, :]


_unpack_tc = pl.pallas_call(
    _unpack_tc_kernel,
    grid=(BATCH // BLK,),
    in_specs=[pl.BlockSpec((BLK, D_PAD), lambda j: (j, 0))
              for _ in range(N_DEPTHS)],
    out_specs=[pl.BlockSpec((N_LABELS, BLK), lambda j: (0, j))
               for _ in range(N_DEPTHS)],
    out_shape=[jax.ShapeDtypeStruct((N_LABELS, BATCH), jnp.int8)
               for _ in range(N_DEPTHS)],
)


@jax.jit
def kernel(labels, adversaries):
  a = jnp.pad(adversaries, ((0, 0), (0, 0), (0, 4 * D_PAD - N_LABELS)))
  a = a.reshape(N_DEPTHS, N_LABELS, D_PAD, 4).astype(jnp.uint32)
  w = (a[..., 0] | (a[..., 1] << 8) | (a[..., 2] << 16) | (a[..., 3] << 24))
  tbl = jax.lax.bitcast_convert_type(
      w.reshape(N_DEPTHS * N_LABELS, D_PAD), jnp.int32)
  lab = labels.reshape(BATCH)

  outs = _sc_gather(tbl, lab)
  u8s = _unpack_tc(*outs)
  return tuple(jnp.transpose(u).astype(jnp.bool_) for u in u8s)
